# baseline (speedup 1.0000x reference)
import functools

import jax
import jax.numpy as jnp
from jax import lax
from jax.experimental import pallas as pl
from jax.experimental.pallas import tpu as pltpu
from jax.experimental.pallas import tpu_sc as plsc

N = 10000
E = 160000
IN_FEATS = 256
OUT_FEATS = 64
HEADS = 8
F = HEADS * OUT_FEATS
NEG_SLOPE = 0.2

NC = 2
NS = 16
NW = NC * NS

_MB = 1000


def _mm_body(x_ref, ws_ref, wd_ref, fs_ref, fd_ref):
    x = x_ref[...]
    fs_ref[...] = jnp.dot(x, ws_ref[...], preferred_element_type=jnp.float32)
    fd_ref[...] = jnp.dot(x, wd_ref[...], preferred_element_type=jnp.float32)


_mm = pl.pallas_call(
    _mm_body,
    grid=(N // _MB,),
    in_specs=[
        pl.BlockSpec((_MB, IN_FEATS), lambda i: (i, 0)),
        pl.BlockSpec((IN_FEATS, F), lambda i: (0, 0)),
        pl.BlockSpec((IN_FEATS, F), lambda i: (0, 0)),
    ],
    out_specs=[
        pl.BlockSpec((_MB, F), lambda i: (i, 0)),
        pl.BlockSpec((_MB, F), lambda i: (i, 0)),
    ],
    out_shape=[jax.ShapeDtypeStruct((N, F), jnp.float32)] * 2,
)

_GB = 40
_EPW = E // NW

@functools.cache
def _sc_mesh():
    return plsc.VectorSubcoreMesh(
        core_axis_name="c", subcore_axis_name="s", num_cores=NC, num_subcores=NS)


_GNI = _EPW // _GB


def _fused_body(fs_hbm, fd_hbm, src_hbm, dst_hbm, attn_hbm, z128_hbm,
                msgf_hbm,
                si0, si1, di0, di1, abuf0, abuf1, bbuf0, bbuf1,
                mb0, mb1, exrow, attnv,
                sga0, sga1, sgb0, sgb1, swm0, swm1, swe):
    wid = lax.axis_index("s") * NC + lax.axis_index("c")
    base = wid * _EPW
    si = (si0, si1)
    di = (di0, di1)
    abuf = (abuf0, abuf1)
    bbuf = (bbuf0, bbuf1)
    mb = (mb0, mb1)
    sga = (sga0, sga1)
    sgb = (sgb0, sgb1)
    swm = (swm0, swm1)

    pltpu.sync_copy(attn_hbm, attnv)
    pltpu.sync_copy(z128_hbm.at[pl.ds(0, _GB)], exrow)

    lane = lax.iota(jnp.int32, 16)

    def g_issue(i, b):
        off = base + i * _GB
        pltpu.sync_copy(src_hbm.at[pl.ds(off, _GB)], si[b])
        pltpu.sync_copy(dst_hbm.at[pl.ds(off, _GB)], di[b])
        pltpu.async_copy(fs_hbm.at[si[b]], abuf[b], sga[b])
        pltpu.async_copy(fd_hbm.at[di[b]], bbuf[b], sgb[b])

    def g_wait(b):
        pltpu.make_async_copy(fs_hbm.at[si[b]], abuf[b], sga[b]).wait()
        pltpu.make_async_copy(fd_hbm.at[di[b]], bbuf[b], sgb[b]).wait()

    def compute(i, b):
        @pl.when(i >= 1)
        def _():
            pltpu.make_async_copy(exrow, msgf_hbm.at[pl.ds(0, _GB)],
                                  swe).wait()

        @plsc.parallel_loop(0, _GB, unroll=8)
        def edge(e):
            accs = []
            for h in range(8):
                acc = None
                for jj in range(4):
                    col = 64 * h + 16 * jj
                    a = abuf[b][e, pl.ds(col, 16)]
                    d = bbuf[b][e, pl.ds(col, 16)]
                    t = a + d
                    t = jnp.maximum(t, NEG_SLOPE * t)
                    p = t * attnv[pl.ds(col, 16)]
                    acc = p if acc is None else acc + p
                for sh in (8, 4, 2, 1):
                    acc = acc + acc.at[lane ^ sh].get(
                        mode="promise_in_bounds")
                accs.append(acc)
            merged = jnp.zeros((16,), jnp.float32)
            for h in range(8):
                merged = merged + jnp.where(lane == h, accs[h], 0.0)
            ex = jnp.exp(merged)
            exrow[e, pl.ds(0, 16)] = ex
            for h in range(8):
                bc = ex.at[jnp.full((16,), h, jnp.int32)].get(
                    mode="promise_in_bounds")
                for jj in range(4):
                    col = 64 * h + 16 * jj
                    m = abuf[b][e, pl.ds(col, 16)] * bc
                    mb[b][col // 128, e, pl.ds(col % 128, 16)] = m

    def w_issue(i, b):
        off = base + i * _GB
        for c in range(4):
            pltpu.async_copy(mb[b].at[c], msgf_hbm.at[pl.ds(c * E + off, _GB)],
                             swm[b])
        pltpu.async_copy(exrow, msgf_hbm.at[pl.ds(4 * E + off, _GB)], swe)

    def w_wait(b):
        for c in range(4):
            pltpu.make_async_copy(mb[b].at[0], msgf_hbm.at[pl.ds(0, _GB)],
                                  swm[b]).wait()

    g_issue(0, 0)
    g_issue(1, 1)

    def pair(k, carry):
        for b in range(2):
            i = 2 * k + b
            g_wait(b)
            compute(i, b)
            w_issue(i, b)

            @pl.when(i + 2 <= _GNI - 1)
            def _():
                w_wait(b)
                g_issue(i + 2, b)
        return carry

    lax.fori_loop(0, (_GNI - 1) // 2, pair, 0)
    g_wait(0)
    compute(_GNI - 1, 0)
    w_issue(_GNI - 1, 0)
    w_wait(0)
    w_wait(1)
    pltpu.make_async_copy(exrow, msgf_hbm.at[pl.ds(0, _GB)], swe).wait()


@functools.cache
def _fused():
    return pl.kernel(
        _fused_body,
        out_type=jax.ShapeDtypeStruct((5 * E, 128), jnp.float32),
        mesh=_sc_mesh(),
        scratch_types=[
            pltpu.VMEM((_GB,), jnp.int32),
            pltpu.VMEM((_GB,), jnp.int32),
            pltpu.VMEM((_GB,), jnp.int32),
            pltpu.VMEM((_GB,), jnp.int32),
            pltpu.VMEM((_GB, F), jnp.float32),
            pltpu.VMEM((_GB, F), jnp.float32),
            pltpu.VMEM((_GB, F), jnp.float32),
            pltpu.VMEM((_GB, F), jnp.float32),
            pltpu.VMEM((4, _GB, 128), jnp.float32),
            pltpu.VMEM((4, _GB, 128), jnp.float32),
            pltpu.VMEM((_GB, 128), jnp.float32),
            pltpu.VMEM((F,), jnp.float32),
        ] + [pltpu.SemaphoreType.DMA] * 7,
    )

_EB = 2000


_SB = 80
_EPT = E // NS
_ZR = 624
_ZTAIL = N - NS * _ZR


_ZB = 48


_SB4 = 40
_SNI = _EPT // _SB
_SNI4 = (_EPT // 2) // _SB4


def _scatter_body(msgf_hbm, dst_hbm, z128_hbm,
                  unf_hbm, acc, vb0, vb1, ib0, ib1, i40, i41,
                  zb0, zb1, sr0, sr1, sz, sd0, sd1):
    cid = lax.axis_index("c")
    sid = lax.axis_index("s")
    nbase = sid * _ZR
    vb = (vb0, vb1)
    ib = (ib0, ib1)
    ib4 = (i40, i41)
    zb = (zb0, zb1)
    sr = (sr0, sr1)
    sd = (sd0, sd1)

    pltpu.sync_copy(z128_hbm.at[pl.ds(0, _ZB)], zb0)

    def zero_acc():
        for j in range(_ZR // _ZB):
            pltpu.async_copy(zb0, acc.at[pl.ds(nbase + j * _ZB, _ZB)], sz)
        for j in range(_ZR // _ZB):
            pltpu.make_async_copy(zb0, acc.at[pl.ds(nbase, _ZB)], sz).wait()

        @pl.when(sid == NS - 1)
        def _():
            pltpu.sync_copy(zb0.at[pl.ds(0, _ZTAIL)],
                            acc.at[pl.ds(NS * _ZR, _ZTAIL)])

    def drain(plane):
        for j in range(_ZR // _ZB):
            b = j % 2
            if j >= 2:
                pltpu.make_async_copy(
                    zb[b], unf_hbm.at[pl.ds(0, _ZB)], sd[b]).wait()
            pltpu.sync_copy(acc.at[pl.ds(nbase + j * _ZB, _ZB)], zb[b])
            pltpu.async_copy(
                zb[b], unf_hbm.at[pl.ds(plane * N + nbase + j * _ZB, _ZB)],
                sd[b])
        for b in range(2):
            pltpu.make_async_copy(
                zb[b], unf_hbm.at[pl.ds(0, _ZB)], sd[b]).wait()

        @pl.when(sid == NS - 1)
        def _():
            pltpu.sync_copy(acc.at[pl.ds(NS * _ZR, _ZTAIL)],
                            zb0.at[pl.ds(0, _ZTAIL)])
            pltpu.sync_copy(zb0.at[pl.ds(0, _ZTAIL)],
                            unf_hbm.at[pl.ds(plane * N + NS * _ZR, _ZTAIL)])
        pltpu.sync_copy(z128_hbm.at[pl.ds(0, _ZB)], zb0)

    for rep in range(2):
        chunk = cid * 2 + rep
        ebase = chunk * E + sid * _EPT
        zero_acc()
        plsc.subcore_barrier()

        def r_issue(i, b):
            eoff = sid * _EPT + i * _SB
            pltpu.async_copy(dst_hbm.at[pl.ds(eoff, _SB)], ib[b], sr[b])
            pltpu.async_copy(
                msgf_hbm.at[pl.ds(ebase + i * _SB, _SB)], vb[b], sr[b])

        def r_wait(b):
            pltpu.make_async_copy(dst_hbm.at[pl.ds(0, _SB)], ib[b],
                                  sr[b]).wait()
            pltpu.make_async_copy(
                msgf_hbm.at[pl.ds(0, _SB)], vb[b], sr[b]).wait()

        r_issue(0, 0)
        r_issue(1, 1)

        def pair(k, carry):
            for b in range(2):
                i = 2 * k + b
                r_wait(b)
                pltpu.sync_copy(vb[b], acc.at[ib[b]], add=True)

                @pl.when(i + 2 <= _SNI - 1)
                def _():
                    r_issue(i + 2, b)
            return carry

        lax.fori_loop(0, (_SNI - 1) // 2, pair, 0)
        r_wait(0)
        pltpu.sync_copy(vb[0], acc.at[ib[0]], add=True)
        plsc.subcore_barrier()
        drain(chunk)
        plsc.subcore_barrier()

    zero_acc()
    plsc.subcore_barrier()
    ebase4 = 4 * E + cid * (E // 2) + sid * (_EPT // 2)

    eibase4 = cid * (E // 2) + sid * (_EPT // 2)

    def r4_issue(i, b):
        pltpu.async_copy(dst_hbm.at[pl.ds(eibase4 + i * _SB4, _SB4)],
                         ib4[b], sr[b])
        pltpu.async_copy(
            msgf_hbm.at[pl.ds(ebase4 + i * _SB4, _SB4)],
            vb[b].at[pl.ds(0, _SB4)], sr[b])

    def r4_wait(b):
        pltpu.make_async_copy(dst_hbm.at[pl.ds(0, _SB4)], ib4[b],
                              sr[b]).wait()
        pltpu.make_async_copy(
            msgf_hbm.at[pl.ds(0, _SB4)], vb[b].at[pl.ds(0, _SB4)],
            sr[b]).wait()

    r4_issue(0, 0)
    r4_issue(1, 1)

    def pair4(k, carry):
        for b in range(2):
            i = 2 * k + b
            r4_wait(b)
            pltpu.sync_copy(vb[b].at[pl.ds(0, _SB4)], acc.at[ib4[b]],
                            add=True)

            @pl.when(i + 2 <= _SNI4 - 1)
            def _():
                r4_issue(i + 2, b)
        return carry

    lax.fori_loop(0, (_SNI4 - 1) // 2, pair4, 0)
    r4_wait(0)
    pltpu.sync_copy(vb[0].at[pl.ds(0, _SB4)], acc.at[ib4[0]], add=True)
    plsc.subcore_barrier()
    drain(4 + cid)


@functools.cache
def _scatter():
    return pl.kernel(
        _scatter_body,
        out_type=jax.ShapeDtypeStruct((6 * N, 128), jnp.float32),
        mesh=_sc_mesh(),
        scratch_types=[
            pltpu.VMEM_SHARED((N, 128), jnp.float32),
            pltpu.VMEM((_SB, 128), jnp.float32),
            pltpu.VMEM((_SB, 128), jnp.float32),
            pltpu.VMEM((_SB,), jnp.int32),
            pltpu.VMEM((_SB,), jnp.int32),
            pltpu.VMEM((_SB4,), jnp.int32),
            pltpu.VMEM((_SB4,), jnp.int32),
            pltpu.VMEM((_ZB, 128), jnp.float32),
            pltpu.VMEM((_ZB, 128), jnp.float32),
        ] + [pltpu.SemaphoreType.DMA] * 5,
    )

_NB = 2000


def _norm_body(u_ref, x_ref, b_ref, o_ref):
    d = u_ref[4] + u_ref[5]
    inv = 1.0 / (d + 1e-16)
    scale = jnp.dot(inv, x_ref[...], preferred_element_type=jnp.float32)
    u = jnp.concatenate([u_ref[c] for c in range(4)], axis=1)
    o_ref[...] = u * scale + b_ref[...]


_norm = pl.pallas_call(
    _norm_body,
    grid=(N // _NB,),
    in_specs=[
        pl.BlockSpec((6, _NB, 128), lambda i: (0, i, 0)),
        pl.BlockSpec((128, F), lambda i: (0, 0)),
        pl.BlockSpec((1, F), lambda i: (0, 0)),
    ],
    out_specs=pl.BlockSpec((_NB, F), lambda i: (i, 0)),
    out_shape=jax.ShapeDtypeStruct((N, F), jnp.float32),
)


def kernel(feats, edge_index, W_src, W_dst, attn, bias):
    fs, fd = _mm(feats, W_src, W_dst)
    src = edge_index[0]
    dst = edge_index[1]
    z128 = jnp.zeros((_ZR, 128), jnp.float32)
    msgf = _fused()(fs, fd, src, dst, attn.reshape(F), z128)
    unf = _scatter()(msgf, dst, z128)

    head = jnp.arange(F, dtype=jnp.int32) // OUT_FEATS
    sel = head[:, None] == jnp.arange(16, dtype=jnp.int32)[None, :]
    x16 = sel.T.astype(jnp.float32)
    x128 = jnp.concatenate([x16, jnp.zeros((112, F), jnp.float32)], axis=0)
    return _norm(unf.reshape(6, N, 128), x128, bias.reshape(1, F))


# speedup vs baseline: 1.1839x; 1.1839x over previous
import functools

import jax
import jax.numpy as jnp
from jax import lax
from jax.experimental import pallas as pl
from jax.experimental.pallas import tpu as pltpu
from jax.experimental.pallas import tpu_sc as plsc

N = 10000
E = 160000
IN_FEATS = 256
OUT_FEATS = 64
HEADS = 8
F = HEADS * OUT_FEATS
NEG_SLOPE = 0.2

NC = 2
NS = 16
NW = NC * NS

_MB = 1000


def _mm_body(x_ref, ws_ref, wd_ref, fs_ref, fd_ref):
    x = x_ref[...]
    fs_ref[...] = jnp.dot(x, ws_ref[...], preferred_element_type=jnp.float32)
    fd_ref[...] = jnp.dot(x, wd_ref[...], preferred_element_type=jnp.float32)


_mm = pl.pallas_call(
    _mm_body,
    grid=(N // _MB,),
    in_specs=[
        pl.BlockSpec((_MB, IN_FEATS), lambda i: (i, 0)),
        pl.BlockSpec((IN_FEATS, F), lambda i: (0, 0)),
        pl.BlockSpec((IN_FEATS, F), lambda i: (0, 0)),
    ],
    out_specs=[
        pl.BlockSpec((_MB, F), lambda i: (i, 0)),
        pl.BlockSpec((_MB, F), lambda i: (i, 0)),
    ],
    out_shape=[jax.ShapeDtypeStruct((N, F), jnp.float32)] * 2,
)

_GB = 40
_EPW = E // NW

@functools.cache
def _sc_mesh():
    return plsc.VectorSubcoreMesh(
        core_axis_name="c", subcore_axis_name="s", num_cores=NC, num_subcores=NS)


_GNI = _EPW // _GB


def _fused_body(fs_hbm, fd_hbm, src_hbm, dst_hbm, attn_hbm, z128_hbm,
                msgf_hbm,
                si0, si1, di0, di1, abuf0, abuf1, bbuf0, bbuf1,
                mb0, mb1, exrow, attnv,
                sga0, sga1, sgb0, sgb1, swm0, swm1, swe):
    wid = lax.axis_index("s") * NC + lax.axis_index("c")
    base = wid * _EPW
    si = (si0, si1)
    di = (di0, di1)
    abuf = (abuf0, abuf1)
    bbuf = (bbuf0, bbuf1)
    mb = (mb0, mb1)
    sga = (sga0, sga1)
    sgb = (sgb0, sgb1)
    swm = (swm0, swm1)

    pltpu.sync_copy(attn_hbm, attnv)
    pltpu.sync_copy(z128_hbm.at[pl.ds(0, _GB)], exrow)

    lane = lax.iota(jnp.int32, 16)

    def g_issue(i, b):
        off = base + i * _GB
        pltpu.sync_copy(src_hbm.at[pl.ds(off, _GB)], si[b])
        pltpu.sync_copy(dst_hbm.at[pl.ds(off, _GB)], di[b])
        pltpu.async_copy(fs_hbm.at[si[b]], abuf[b], sga[b])
        pltpu.async_copy(fd_hbm.at[di[b]], bbuf[b], sgb[b])

    def g_wait(b):
        pltpu.make_async_copy(fs_hbm.at[si[b]], abuf[b], sga[b]).wait()
        pltpu.make_async_copy(fd_hbm.at[di[b]], bbuf[b], sgb[b]).wait()

    def compute(i, b):
        @pl.when(i >= 1)
        def _():
            pltpu.make_async_copy(exrow, msgf_hbm.at[pl.ds(0, _GB)],
                                  swe).wait()

        @plsc.parallel_loop(0, _GB, unroll=6)
        def edge(e):
            accs = []
            for h in range(8):
                acc = None
                for jj in range(4):
                    col = 64 * h + 16 * jj
                    a = abuf[b][e, pl.ds(col, 16)]
                    d = bbuf[b][e, pl.ds(col, 16)]
                    t = a + d
                    t = jnp.maximum(t, NEG_SLOPE * t)
                    p = t * attnv[pl.ds(col, 16)]
                    acc = p if acc is None else acc + p
                for sh in (8, 4, 2, 1):
                    acc = acc + acc.at[lane ^ sh].get(
                        mode="promise_in_bounds")
                accs.append(acc)
            merged = jnp.zeros((16,), jnp.float32)
            for h in range(8):
                merged = merged + jnp.where(lane == h, accs[h], 0.0)
            ex = jnp.exp(merged)
            exrow[e, pl.ds(0, 16)] = ex
            for h in range(8):
                bc = ex.at[jnp.full((16,), h, jnp.int32)].get(
                    mode="promise_in_bounds")
                for jj in range(4):
                    col = 64 * h + 16 * jj
                    m = abuf[b][e, pl.ds(col, 16)] * bc
                    mb[b][col // 128, e, pl.ds(col % 128, 16)] = m

    def w_issue(i, b):
        off = base + i * _GB
        for c in range(4):
            pltpu.async_copy(mb[b].at[c], msgf_hbm.at[pl.ds(c * E + off, _GB)],
                             swm[b])
        pltpu.async_copy(exrow, msgf_hbm.at[pl.ds(4 * E + off, _GB)], swe)

    def w_wait(b):
        for c in range(4):
            pltpu.make_async_copy(mb[b].at[0], msgf_hbm.at[pl.ds(0, _GB)],
                                  swm[b]).wait()

    g_issue(0, 0)
    g_issue(1, 1)

    def pair(k, carry):
        for b in range(2):
            i = 2 * k + b
            g_wait(b)
            compute(i, b)
            w_issue(i, b)

            @pl.when(i + 2 <= _GNI - 1)
            def _():
                w_wait(b)
                g_issue(i + 2, b)
        return carry

    lax.fori_loop(0, (_GNI - 1) // 2, pair, 0)
    g_wait(0)
    compute(_GNI - 1, 0)
    w_issue(_GNI - 1, 0)
    w_wait(0)
    w_wait(1)
    pltpu.make_async_copy(exrow, msgf_hbm.at[pl.ds(0, _GB)], swe).wait()


@functools.cache
def _fused():
    return pl.kernel(
        _fused_body,
        out_type=jax.ShapeDtypeStruct((5 * E, 128), jnp.float32),
        mesh=_sc_mesh(),
        scratch_types=[
            pltpu.VMEM((_GB,), jnp.int32),
            pltpu.VMEM((_GB,), jnp.int32),
            pltpu.VMEM((_GB,), jnp.int32),
            pltpu.VMEM((_GB,), jnp.int32),
            pltpu.VMEM((_GB, F), jnp.float32),
            pltpu.VMEM((_GB, F), jnp.float32),
            pltpu.VMEM((_GB, F), jnp.float32),
            pltpu.VMEM((_GB, F), jnp.float32),
            pltpu.VMEM((4, _GB, 128), jnp.float32),
            pltpu.VMEM((4, _GB, 128), jnp.float32),
            pltpu.VMEM((_GB, 128), jnp.float32),
            pltpu.VMEM((F,), jnp.float32),
        ] + [pltpu.SemaphoreType.DMA] * 7,
    )

_EB = 2000


_SB = 80
_EPT = E // NS
_ZR = 624
_ZTAIL = N - NS * _ZR


_ZB = 48


_SB4 = 40
_SNI = _EPT // _SB
_SNI4 = (_EPT // 2) // _SB4


def _scatter_body(msgf_hbm, dst_hbm, z128_hbm,
                  unf_hbm, acc, vb0, vb1, ib0, ib1, i40, i41,
                  zb0, zb1, sr0, sr1, sz, sd0, sd1):
    cid = lax.axis_index("c")
    sid = lax.axis_index("s")
    nbase = sid * _ZR
    vb = (vb0, vb1)
    ib = (ib0, ib1)
    ib4 = (i40, i41)
    zb = (zb0, zb1)
    sr = (sr0, sr1)
    sd = (sd0, sd1)

    pltpu.sync_copy(z128_hbm.at[pl.ds(0, _ZB)], zb0)

    def zero_acc():
        for j in range(_ZR // _ZB):
            pltpu.async_copy(zb0, acc.at[pl.ds(nbase + j * _ZB, _ZB)], sz)
        for j in range(_ZR // _ZB):
            pltpu.make_async_copy(zb0, acc.at[pl.ds(nbase, _ZB)], sz).wait()

        @pl.when(sid == NS - 1)
        def _():
            pltpu.sync_copy(zb0.at[pl.ds(0, _ZTAIL)],
                            acc.at[pl.ds(NS * _ZR, _ZTAIL)])

    def drain(plane):
        for j in range(_ZR // _ZB):
            b = j % 2
            if j >= 2:
                pltpu.make_async_copy(
                    zb[b], unf_hbm.at[pl.ds(0, _ZB)], sd[b]).wait()
            pltpu.sync_copy(acc.at[pl.ds(nbase + j * _ZB, _ZB)], zb[b])
            pltpu.async_copy(
                zb[b], unf_hbm.at[pl.ds(plane * N + nbase + j * _ZB, _ZB)],
                sd[b])
        for b in range(2):
            pltpu.make_async_copy(
                zb[b], unf_hbm.at[pl.ds(0, _ZB)], sd[b]).wait()

        @pl.when(sid == NS - 1)
        def _():
            pltpu.sync_copy(acc.at[pl.ds(NS * _ZR, _ZTAIL)],
                            zb0.at[pl.ds(0, _ZTAIL)])
            pltpu.sync_copy(zb0.at[pl.ds(0, _ZTAIL)],
                            unf_hbm.at[pl.ds(plane * N + NS * _ZR, _ZTAIL)])
        pltpu.sync_copy(z128_hbm.at[pl.ds(0, _ZB)], zb0)

    for rep in range(2):
        chunk = cid * 2 + rep
        ebase = chunk * E + sid * _EPT
        zero_acc()
        plsc.subcore_barrier()

        def r_issue(i, b):
            eoff = sid * _EPT + i * _SB
            pltpu.async_copy(dst_hbm.at[pl.ds(eoff, _SB)], ib[b], sr[b])
            pltpu.async_copy(
                msgf_hbm.at[pl.ds(ebase + i * _SB, _SB)], vb[b], sr[b])

        def r_wait(b):
            pltpu.make_async_copy(dst_hbm.at[pl.ds(0, _SB)], ib[b],
                                  sr[b]).wait()
            pltpu.make_async_copy(
                msgf_hbm.at[pl.ds(0, _SB)], vb[b], sr[b]).wait()

        r_issue(0, 0)
        r_issue(1, 1)

        def pair(k, carry):
            for b in range(2):
                i = 2 * k + b
                r_wait(b)
                pltpu.sync_copy(vb[b], acc.at[ib[b]], add=True)

                @pl.when(i + 2 <= _SNI - 1)
                def _():
                    r_issue(i + 2, b)
            return carry

        lax.fori_loop(0, (_SNI - 1) // 2, pair, 0)
        r_wait(0)
        pltpu.sync_copy(vb[0], acc.at[ib[0]], add=True)
        plsc.subcore_barrier()
        drain(chunk)
        plsc.subcore_barrier()

    zero_acc()
    plsc.subcore_barrier()
    ebase4 = 4 * E + cid * (E // 2) + sid * (_EPT // 2)

    eibase4 = cid * (E // 2) + sid * (_EPT // 2)

    def r4_issue(i, b):
        pltpu.async_copy(dst_hbm.at[pl.ds(eibase4 + i * _SB4, _SB4)],
                         ib4[b], sr[b])
        pltpu.async_copy(
            msgf_hbm.at[pl.ds(ebase4 + i * _SB4, _SB4)],
            vb[b].at[pl.ds(0, _SB4)], sr[b])

    def r4_wait(b):
        pltpu.make_async_copy(dst_hbm.at[pl.ds(0, _SB4)], ib4[b],
                              sr[b]).wait()
        pltpu.make_async_copy(
            msgf_hbm.at[pl.ds(0, _SB4)], vb[b].at[pl.ds(0, _SB4)],
            sr[b]).wait()

    r4_issue(0, 0)
    r4_issue(1, 1)

    def pair4(k, carry):
        for b in range(2):
            i = 2 * k + b
            r4_wait(b)
            pltpu.sync_copy(vb[b].at[pl.ds(0, _SB4)], acc.at[ib4[b]],
                            add=True)

            @pl.when(i + 2 <= _SNI4 - 1)
            def _():
                r4_issue(i + 2, b)
        return carry

    lax.fori_loop(0, (_SNI4 - 1) // 2, pair4, 0)
    r4_wait(0)
    pltpu.sync_copy(vb[0].at[pl.ds(0, _SB4)], acc.at[ib4[0]], add=True)
    plsc.subcore_barrier()
    drain(4 + cid)


@functools.cache
def _scatter():
    return pl.kernel(
        _scatter_body,
        out_type=jax.ShapeDtypeStruct((6 * N, 128), jnp.float32),
        mesh=_sc_mesh(),
        scratch_types=[
            pltpu.VMEM_SHARED((N, 128), jnp.float32),
            pltpu.VMEM((_SB, 128), jnp.float32),
            pltpu.VMEM((_SB, 128), jnp.float32),
            pltpu.VMEM((_SB,), jnp.int32),
            pltpu.VMEM((_SB,), jnp.int32),
            pltpu.VMEM((_SB4,), jnp.int32),
            pltpu.VMEM((_SB4,), jnp.int32),
            pltpu.VMEM((_ZB, 128), jnp.float32),
            pltpu.VMEM((_ZB, 128), jnp.float32),
        ] + [pltpu.SemaphoreType.DMA] * 5,
    )

_NB = 2000


def _norm_body(u_ref, x_ref, b_ref, o_ref):
    d = u_ref[4] + u_ref[5]
    inv = 1.0 / (d + 1e-16)
    scale = jnp.dot(inv, x_ref[...], preferred_element_type=jnp.float32)
    u = jnp.concatenate([u_ref[c] for c in range(4)], axis=1)
    o_ref[...] = u * scale + b_ref[...]


_norm = pl.pallas_call(
    _norm_body,
    grid=(N // _NB,),
    in_specs=[
        pl.BlockSpec((6, _NB, 128), lambda i: (0, i, 0)),
        pl.BlockSpec((128, F), lambda i: (0, 0)),
        pl.BlockSpec((1, F), lambda i: (0, 0)),
    ],
    out_specs=pl.BlockSpec((_NB, F), lambda i: (i, 0)),
    out_shape=jax.ShapeDtypeStruct((N, F), jnp.float32),
)


def kernel(feats, edge_index, W_src, W_dst, attn, bias):
    fs, fd = _mm(feats, W_src, W_dst)
    src = edge_index[0]
    dst = edge_index[1]
    z128 = jnp.zeros((_ZR, 128), jnp.float32)
    msgf = _fused()(fs, fd, src, dst, attn.reshape(F), z128)
    unf = _scatter()(msgf, dst, z128)

    head = jnp.arange(F, dtype=jnp.int32) // OUT_FEATS
    sel = head[:, None] == jnp.arange(16, dtype=jnp.int32)[None, :]
    x16 = sel.T.astype(jnp.float32)
    x128 = jnp.concatenate([x16, jnp.zeros((112, F), jnp.float32)], axis=0)
    return _norm(unf.reshape(6, N, 128), x128, bias.reshape(1, F))


# speedup vs baseline: 1.4428x; 1.2187x over previous
import functools

import jax
import jax.numpy as jnp
from jax import lax
from jax.experimental import pallas as pl
from jax.experimental.pallas import tpu as pltpu
from jax.experimental.pallas import tpu_sc as plsc

N = 10000
E = 160000
IN_FEATS = 256
OUT_FEATS = 64
HEADS = 8
F = HEADS * OUT_FEATS
NEG_SLOPE = 0.2

NC = 2
NS = 16
NW = NC * NS

_MB = 1000


def _mm_body(x_ref, ws_ref, wd_ref, fs_ref, fd_ref):
    x = x_ref[...]
    fs_ref[...] = jnp.dot(x, ws_ref[...], preferred_element_type=jnp.float32)
    fd_ref[...] = jnp.dot(x, wd_ref[...], preferred_element_type=jnp.float32)


_mm = pl.pallas_call(
    _mm_body,
    grid=(N // _MB,),
    in_specs=[
        pl.BlockSpec((_MB, IN_FEATS), lambda i: (i, 0)),
        pl.BlockSpec((IN_FEATS, F), lambda i: (0, 0)),
        pl.BlockSpec((IN_FEATS, F), lambda i: (0, 0)),
    ],
    out_specs=[
        pl.BlockSpec((_MB, F), lambda i: (i, 0)),
        pl.BlockSpec((_MB, F), lambda i: (i, 0)),
    ],
    out_shape=[jax.ShapeDtypeStruct((N, F), jnp.float32)] * 2,
)

_GB = 40
_EPW = E // NW

@functools.cache
def _sc_mesh():
    return plsc.VectorSubcoreMesh(
        core_axis_name="c", subcore_axis_name="s", num_cores=NC, num_subcores=NS)


_GNI = _EPW // _GB


def _fused_body(fs_hbm, fd_hbm, src_hbm, dst_hbm, attn_hbm, z128_hbm,
                msgf_hbm,
                si0, si1, di0, di1, abuf0, abuf1, bbuf0, bbuf1,
                mb0, mb1, exrow, attnv,
                sga0, sga1, sgb0, sgb1, swm0, swm1, swe):
    wid = lax.axis_index("s") * NC + lax.axis_index("c")
    base = wid * _EPW
    si = (si0, si1)
    di = (di0, di1)
    abuf = (abuf0, abuf1)
    bbuf = (bbuf0, bbuf1)
    mb = (mb0, mb1)
    sga = (sga0, sga1)
    sgb = (sgb0, sgb1)
    swm = (swm0, swm1)

    pltpu.sync_copy(attn_hbm, attnv)
    pltpu.sync_copy(z128_hbm.at[pl.ds(0, _GB)], exrow)

    lane = lax.iota(jnp.int32, 16)
    attn_vecs = [attnv[pl.ds(16 * j, 16)] for j in range(F // 16)]

    def g_issue(i, b):
        off = base + i * _GB
        pltpu.sync_copy(src_hbm.at[pl.ds(off, _GB)], si[b])
        pltpu.sync_copy(dst_hbm.at[pl.ds(off, _GB)], di[b])
        pltpu.async_copy(fs_hbm.at[si[b]], abuf[b], sga[b])
        pltpu.async_copy(fd_hbm.at[di[b]], bbuf[b], sgb[b])

    def g_wait(b):
        pltpu.make_async_copy(fs_hbm.at[si[b]], abuf[b], sga[b]).wait()
        pltpu.make_async_copy(fd_hbm.at[di[b]], bbuf[b], sgb[b]).wait()

    def compute(i, b):
        @pl.when(i >= 1)
        def _():
            pltpu.make_async_copy(exrow, msgf_hbm.at[pl.ds(0, _GB)],
                                  swe).wait()

        @plsc.parallel_loop(0, _GB, unroll=4)
        def edge(e):
            accs = []
            for h in range(8):
                acc = None
                for jj in range(4):
                    col = 64 * h + 16 * jj
                    a = abuf[b][e, pl.ds(col, 16)]
                    d = bbuf[b][e, pl.ds(col, 16)]
                    t = a + d
                    t = jnp.maximum(t, NEG_SLOPE * t)
                    p = t * attn_vecs[4 * h + jj]
                    acc = p if acc is None else acc + p
                for sh in (8, 4, 2, 1):
                    acc = acc + acc.at[lane ^ sh].get(
                        mode="promise_in_bounds")
                accs.append(acc)
            merged = jnp.zeros((16,), jnp.float32)
            for h in range(8):
                merged = merged + jnp.where(lane == h, accs[h], 0.0)
            ex = jnp.exp(merged)
            exrow[e, pl.ds(0, 16)] = ex
            for h in range(8):
                bc = ex.at[jnp.full((16,), h, jnp.int32)].get(
                    mode="promise_in_bounds")
                for jj in range(4):
                    col = 64 * h + 16 * jj
                    m = abuf[b][e, pl.ds(col, 16)] * bc
                    mb[b][col // 128, e, pl.ds(col % 128, 16)] = m

    def w_issue(i, b):
        off = base + i * _GB
        for c in range(4):
            pltpu.async_copy(mb[b].at[c], msgf_hbm.at[pl.ds(c * E + off, _GB)],
                             swm[b])
        pltpu.async_copy(exrow, msgf_hbm.at[pl.ds(4 * E + off, _GB)], swe)

    def w_wait(b):
        for c in range(4):
            pltpu.make_async_copy(mb[b].at[0], msgf_hbm.at[pl.ds(0, _GB)],
                                  swm[b]).wait()

    g_issue(0, 0)
    g_issue(1, 1)

    def pair(k, carry):
        for b in range(2):
            i = 2 * k + b
            g_wait(b)
            compute(i, b)
            w_issue(i, b)

            @pl.when(i + 2 <= _GNI - 1)
            def _():
                w_wait(b)
                g_issue(i + 2, b)
        return carry

    lax.fori_loop(0, (_GNI - 1) // 2, pair, 0)
    g_wait(0)
    compute(_GNI - 1, 0)
    w_issue(_GNI - 1, 0)
    w_wait(0)
    w_wait(1)
    pltpu.make_async_copy(exrow, msgf_hbm.at[pl.ds(0, _GB)], swe).wait()


@functools.cache
def _fused():
    return pl.kernel(
        _fused_body,
        out_type=jax.ShapeDtypeStruct((5 * E, 128), jnp.float32),
        mesh=_sc_mesh(),
        scratch_types=[
            pltpu.VMEM((_GB,), jnp.int32),
            pltpu.VMEM((_GB,), jnp.int32),
            pltpu.VMEM((_GB,), jnp.int32),
            pltpu.VMEM((_GB,), jnp.int32),
            pltpu.VMEM((_GB, F), jnp.float32),
            pltpu.VMEM((_GB, F), jnp.float32),
            pltpu.VMEM((_GB, F), jnp.float32),
            pltpu.VMEM((_GB, F), jnp.float32),
            pltpu.VMEM((4, _GB, 128), jnp.float32),
            pltpu.VMEM((4, _GB, 128), jnp.float32),
            pltpu.VMEM((_GB, 128), jnp.float32),
            pltpu.VMEM((F,), jnp.float32),
        ] + [pltpu.SemaphoreType.DMA] * 7,
    )

_EB = 2000


_SB = 80
_EPT = E // NS
_ZR = 624
_ZTAIL = N - NS * _ZR


_ZB = 48


_SB4 = 40
_SNI = _EPT // _SB
_SNI4 = (_EPT // 2) // _SB4


def _scatter_body(msgf_hbm, dst_hbm, z128_hbm,
                  unf_hbm, acc, vb0, vb1, ib0, ib1, i40, i41,
                  zb0, zb1, sr0, sr1, sz, sd0, sd1):
    cid = lax.axis_index("c")
    sid = lax.axis_index("s")
    nbase = sid * _ZR
    vb = (vb0, vb1)
    ib = (ib0, ib1)
    ib4 = (i40, i41)
    zb = (zb0, zb1)
    sr = (sr0, sr1)
    sd = (sd0, sd1)

    pltpu.sync_copy(z128_hbm.at[pl.ds(0, _ZB)], zb0)

    def zero_acc():
        for j in range(_ZR // _ZB):
            pltpu.async_copy(zb0, acc.at[pl.ds(nbase + j * _ZB, _ZB)], sz)
        for j in range(_ZR // _ZB):
            pltpu.make_async_copy(zb0, acc.at[pl.ds(nbase, _ZB)], sz).wait()

        @pl.when(sid == NS - 1)
        def _():
            pltpu.sync_copy(zb0.at[pl.ds(0, _ZTAIL)],
                            acc.at[pl.ds(NS * _ZR, _ZTAIL)])

    def drain(plane):
        for j in range(_ZR // _ZB):
            b = j % 2
            if j >= 2:
                pltpu.make_async_copy(
                    zb[b], unf_hbm.at[pl.ds(0, _ZB)], sd[b]).wait()
            pltpu.sync_copy(acc.at[pl.ds(nbase + j * _ZB, _ZB)], zb[b])
            pltpu.async_copy(
                zb[b], unf_hbm.at[pl.ds(plane * N + nbase + j * _ZB, _ZB)],
                sd[b])
        for b in range(2):
            pltpu.make_async_copy(
                zb[b], unf_hbm.at[pl.ds(0, _ZB)], sd[b]).wait()

        @pl.when(sid == NS - 1)
        def _():
            pltpu.sync_copy(acc.at[pl.ds(NS * _ZR, _ZTAIL)],
                            zb0.at[pl.ds(0, _ZTAIL)])
            pltpu.sync_copy(zb0.at[pl.ds(0, _ZTAIL)],
                            unf_hbm.at[pl.ds(plane * N + NS * _ZR, _ZTAIL)])
        pltpu.sync_copy(z128_hbm.at[pl.ds(0, _ZB)], zb0)

    for rep in range(2):
        chunk = cid * 2 + rep
        ebase = chunk * E + sid * _EPT
        zero_acc()
        plsc.subcore_barrier()

        def r_issue(i, b):
            eoff = sid * _EPT + i * _SB
            pltpu.async_copy(dst_hbm.at[pl.ds(eoff, _SB)], ib[b], sr[b])
            pltpu.async_copy(
                msgf_hbm.at[pl.ds(ebase + i * _SB, _SB)], vb[b], sr[b])

        def r_wait(b):
            pltpu.make_async_copy(dst_hbm.at[pl.ds(0, _SB)], ib[b],
                                  sr[b]).wait()
            pltpu.make_async_copy(
                msgf_hbm.at[pl.ds(0, _SB)], vb[b], sr[b]).wait()

        r_issue(0, 0)
        r_issue(1, 1)

        def pair(k, carry):
            for b in range(2):
                i = 2 * k + b
                r_wait(b)
                pltpu.sync_copy(vb[b], acc.at[ib[b]], add=True)

                @pl.when(i + 2 <= _SNI - 1)
                def _():
                    r_issue(i + 2, b)
            return carry

        lax.fori_loop(0, (_SNI - 1) // 2, pair, 0)
        r_wait(0)
        pltpu.sync_copy(vb[0], acc.at[ib[0]], add=True)
        plsc.subcore_barrier()
        drain(chunk)
        plsc.subcore_barrier()

    zero_acc()
    plsc.subcore_barrier()
    ebase4 = 4 * E + cid * (E // 2) + sid * (_EPT // 2)

    eibase4 = cid * (E // 2) + sid * (_EPT // 2)

    def r4_issue(i, b):
        pltpu.async_copy(dst_hbm.at[pl.ds(eibase4 + i * _SB4, _SB4)],
                         ib4[b], sr[b])
        pltpu.async_copy(
            msgf_hbm.at[pl.ds(ebase4 + i * _SB4, _SB4)],
            vb[b].at[pl.ds(0, _SB4)], sr[b])

    def r4_wait(b):
        pltpu.make_async_copy(dst_hbm.at[pl.ds(0, _SB4)], ib4[b],
                              sr[b]).wait()
        pltpu.make_async_copy(
            msgf_hbm.at[pl.ds(0, _SB4)], vb[b].at[pl.ds(0, _SB4)],
            sr[b]).wait()

    r4_issue(0, 0)
    r4_issue(1, 1)

    def pair4(k, carry):
        for b in range(2):
            i = 2 * k + b
            r4_wait(b)
            pltpu.sync_copy(vb[b].at[pl.ds(0, _SB4)], acc.at[ib4[b]],
                            add=True)

            @pl.when(i + 2 <= _SNI4 - 1)
            def _():
                r4_issue(i + 2, b)
        return carry

    lax.fori_loop(0, (_SNI4 - 1) // 2, pair4, 0)
    r4_wait(0)
    pltpu.sync_copy(vb[0].at[pl.ds(0, _SB4)], acc.at[ib4[0]], add=True)
    plsc.subcore_barrier()
    drain(4 + cid)


@functools.cache
def _scatter():
    return pl.kernel(
        _scatter_body,
        out_type=jax.ShapeDtypeStruct((6 * N, 128), jnp.float32),
        mesh=_sc_mesh(),
        scratch_types=[
            pltpu.VMEM_SHARED((N, 128), jnp.float32),
            pltpu.VMEM((_SB, 128), jnp.float32),
            pltpu.VMEM((_SB, 128), jnp.float32),
            pltpu.VMEM((_SB,), jnp.int32),
            pltpu.VMEM((_SB,), jnp.int32),
            pltpu.VMEM((_SB4,), jnp.int32),
            pltpu.VMEM((_SB4,), jnp.int32),
            pltpu.VMEM((_ZB, 128), jnp.float32),
            pltpu.VMEM((_ZB, 128), jnp.float32),
        ] + [pltpu.SemaphoreType.DMA] * 5,
    )

_NB = 2000


def _norm_body(u_ref, x_ref, b_ref, o_ref):
    d = u_ref[4] + u_ref[5]
    inv = 1.0 / (d + 1e-16)
    scale = jnp.dot(inv, x_ref[...], preferred_element_type=jnp.float32)
    u = jnp.concatenate([u_ref[c] for c in range(4)], axis=1)
    o_ref[...] = u * scale + b_ref[...]


_norm = pl.pallas_call(
    _norm_body,
    grid=(N // _NB,),
    in_specs=[
        pl.BlockSpec((6, _NB, 128), lambda i: (0, i, 0)),
        pl.BlockSpec((128, F), lambda i: (0, 0)),
        pl.BlockSpec((1, F), lambda i: (0, 0)),
    ],
    out_specs=pl.BlockSpec((_NB, F), lambda i: (i, 0)),
    out_shape=jax.ShapeDtypeStruct((N, F), jnp.float32),
)


def kernel(feats, edge_index, W_src, W_dst, attn, bias):
    fs, fd = _mm(feats, W_src, W_dst)
    src = edge_index[0]
    dst = edge_index[1]
    z128 = jnp.zeros((_ZR, 128), jnp.float32)
    msgf = _fused()(fs, fd, src, dst, attn.reshape(F), z128)
    unf = _scatter()(msgf, dst, z128)

    head = jnp.arange(F, dtype=jnp.int32) // OUT_FEATS
    sel = head[:, None] == jnp.arange(16, dtype=jnp.int32)[None, :]
    x16 = sel.T.astype(jnp.float32)
    x128 = jnp.concatenate([x16, jnp.zeros((112, F), jnp.float32)], axis=0)
    return _norm(unf.reshape(6, N, 128), x128, bias.reshape(1, F))


# speedup vs baseline: 1.4432x; 1.0002x over previous
import functools

import jax
import jax.numpy as jnp
from jax import lax
from jax.experimental import pallas as pl
from jax.experimental.pallas import tpu as pltpu
from jax.experimental.pallas import tpu_sc as plsc

N = 10000
E = 160000
IN_FEATS = 256
OUT_FEATS = 64
HEADS = 8
F = HEADS * OUT_FEATS
NEG_SLOPE = 0.2

NC = 2
NS = 16
NW = NC * NS

_MB = 1000


def _mm_body(x_ref, ws_ref, wd_ref, fs_ref, fd_ref):
    x = x_ref[...]
    fs_ref[...] = jnp.dot(x, ws_ref[...], preferred_element_type=jnp.float32)
    fd_ref[...] = jnp.dot(x, wd_ref[...], preferred_element_type=jnp.float32)


_mm = pl.pallas_call(
    _mm_body,
    grid=(N // _MB,),
    in_specs=[
        pl.BlockSpec((_MB, IN_FEATS), lambda i: (i, 0)),
        pl.BlockSpec((IN_FEATS, F), lambda i: (0, 0)),
        pl.BlockSpec((IN_FEATS, F), lambda i: (0, 0)),
    ],
    out_specs=[
        pl.BlockSpec((_MB, F), lambda i: (i, 0)),
        pl.BlockSpec((_MB, F), lambda i: (i, 0)),
    ],
    out_shape=[jax.ShapeDtypeStruct((N, F), jnp.float32)] * 2,
)

_GB = 40
_EPW = E // NW

@functools.cache
def _sc_mesh():
    return plsc.VectorSubcoreMesh(
        core_axis_name="c", subcore_axis_name="s", num_cores=NC, num_subcores=NS)


_GNI = _EPW // _GB


def _fused_body(fs_hbm, fd_hbm, src_hbm, dst_hbm, attn_hbm, z128_hbm,
                msgf_hbm,
                si0, si1, di0, di1, abuf0, abuf1, bbuf0, bbuf1,
                mb0, mb1, exrow, attnv,
                sga0, sga1, sgb0, sgb1, swm0, swm1, swe):
    wid = lax.axis_index("s") * NC + lax.axis_index("c")
    base = wid * _EPW
    si = (si0, si1)
    di = (di0, di1)
    abuf = (abuf0, abuf1)
    bbuf = (bbuf0, bbuf1)
    mb = (mb0, mb1)
    sga = (sga0, sga1)
    sgb = (sgb0, sgb1)
    swm = (swm0, swm1)

    pltpu.sync_copy(attn_hbm, attnv)
    pltpu.sync_copy(z128_hbm.at[pl.ds(0, _GB)], exrow)

    lane = lax.iota(jnp.int32, 16)
    attn_vecs = [attnv[pl.ds(16 * j, 16)] for j in range(F // 16)]

    def g_issue(i, b):
        off = base + i * _GB
        pltpu.sync_copy(src_hbm.at[pl.ds(off, _GB)], si[b])
        pltpu.sync_copy(dst_hbm.at[pl.ds(off, _GB)], di[b])
        pltpu.async_copy(fs_hbm.at[si[b]], abuf[b], sga[b])
        pltpu.async_copy(fd_hbm.at[di[b]], bbuf[b], sgb[b])

    def g_wait(b):
        pltpu.make_async_copy(fs_hbm.at[si[b]], abuf[b], sga[b]).wait()
        pltpu.make_async_copy(fd_hbm.at[di[b]], bbuf[b], sgb[b]).wait()

    def compute(i, b):
        @pl.when(i >= 1)
        def _():
            pltpu.make_async_copy(exrow, msgf_hbm.at[pl.ds(0, _GB)],
                                  swe).wait()

        @plsc.parallel_loop(0, _GB, unroll=4)
        def edge(e):
            accs = []
            for h in range(8):
                acc = None
                for jj in range(4):
                    col = 64 * h + 16 * jj
                    a = abuf[b][e, pl.ds(col, 16)]
                    d = bbuf[b][e, pl.ds(col, 16)]
                    t = a + d
                    t = jnp.maximum(t, NEG_SLOPE * t)
                    p = t * attn_vecs[4 * h + jj]
                    acc = p if acc is None else acc + p
                for sh in (8, 4, 2, 1):
                    acc = acc + acc.at[lane ^ sh].get(
                        mode="promise_in_bounds")
                accs.append(acc)
            merged = jnp.zeros((16,), jnp.float32)
            for h in range(8):
                merged = merged + jnp.where(lane == h, accs[h], 0.0)
            ex = jnp.exp(merged)
            exrow[e, pl.ds(0, 16)] = ex
            for h in range(8):
                bc = ex.at[jnp.full((16,), h, jnp.int32)].get(
                    mode="promise_in_bounds")
                for jj in range(4):
                    col = 64 * h + 16 * jj
                    m = abuf[b][e, pl.ds(col, 16)] * bc
                    mb[b][col // 128, e, pl.ds(col % 128, 16)] = m

    def w_issue(i, b):
        off = base + i * _GB
        for c in range(4):
            pltpu.async_copy(mb[b].at[c], msgf_hbm.at[pl.ds(c * E + off, _GB)],
                             swm[b])
        pltpu.async_copy(exrow, msgf_hbm.at[pl.ds(4 * E + off, _GB)], swe)

    def w_wait(b):
        for c in range(4):
            pltpu.make_async_copy(mb[b].at[0], msgf_hbm.at[pl.ds(0, _GB)],
                                  swm[b]).wait()

    g_issue(0, 0)
    g_issue(1, 1)

    def pair(k, carry):
        for b in range(2):
            i = 2 * k + b
            g_wait(b)
            compute(i, b)
            w_issue(i, b)

            @pl.when(i + 2 <= _GNI - 1)
            def _():
                w_wait(b)
                g_issue(i + 2, b)
        return carry

    lax.fori_loop(0, (_GNI - 1) // 2, pair, 0)
    g_wait(0)
    compute(_GNI - 1, 0)
    w_issue(_GNI - 1, 0)
    w_wait(0)
    w_wait(1)
    pltpu.make_async_copy(exrow, msgf_hbm.at[pl.ds(0, _GB)], swe).wait()


@functools.cache
def _fused():
    return pl.kernel(
        _fused_body,
        out_type=jax.ShapeDtypeStruct((5 * E, 128), jnp.float32),
        mesh=_sc_mesh(),
        scratch_types=[
            pltpu.VMEM((_GB,), jnp.int32),
            pltpu.VMEM((_GB,), jnp.int32),
            pltpu.VMEM((_GB,), jnp.int32),
            pltpu.VMEM((_GB,), jnp.int32),
            pltpu.VMEM((_GB, F), jnp.float32),
            pltpu.VMEM((_GB, F), jnp.float32),
            pltpu.VMEM((_GB, F), jnp.float32),
            pltpu.VMEM((_GB, F), jnp.float32),
            pltpu.VMEM((4, _GB, 128), jnp.float32),
            pltpu.VMEM((4, _GB, 128), jnp.float32),
            pltpu.VMEM((_GB, 128), jnp.float32),
            pltpu.VMEM((F,), jnp.float32),
        ] + [pltpu.SemaphoreType.DMA] * 7,
    )

_EB = 2000


_SB = 80
_EPT = E // NS
_ZR = 624
_ZTAIL = N - NS * _ZR


_ZB = 48


_SB4 = 40
_SNI = _EPT // _SB
_SNI4 = (_EPT // 2) // _SB4


def _scatter_body(msgf_hbm, dst_hbm, z128_hbm,
                  unf_hbm, acc, vb0, vb1, ib0, ib1, i40, i41,
                  zb0, zb1, sr0, sr1, sz, sd0, sd1, sa0, sa1):
    cid = lax.axis_index("c")
    sid = lax.axis_index("s")
    nbase = sid * _ZR
    vb = (vb0, vb1)
    ib = (ib0, ib1)
    ib4 = (i40, i41)
    zb = (zb0, zb1)
    sr = (sr0, sr1)
    sd = (sd0, sd1)
    sa = (sa0, sa1)

    pltpu.sync_copy(z128_hbm.at[pl.ds(0, _ZB)], zb0)

    def zero_acc():
        for j in range(_ZR // _ZB):
            pltpu.async_copy(zb0, acc.at[pl.ds(nbase + j * _ZB, _ZB)], sz)
        for j in range(_ZR // _ZB):
            pltpu.make_async_copy(zb0, acc.at[pl.ds(nbase, _ZB)], sz).wait()

        @pl.when(sid == NS - 1)
        def _():
            pltpu.sync_copy(zb0.at[pl.ds(0, _ZTAIL)],
                            acc.at[pl.ds(NS * _ZR, _ZTAIL)])

    def drain(plane):
        for j in range(_ZR // _ZB):
            b = j % 2
            if j >= 2:
                pltpu.make_async_copy(
                    zb[b], unf_hbm.at[pl.ds(0, _ZB)], sd[b]).wait()
            pltpu.sync_copy(acc.at[pl.ds(nbase + j * _ZB, _ZB)], zb[b])
            pltpu.async_copy(
                zb[b], unf_hbm.at[pl.ds(plane * N + nbase + j * _ZB, _ZB)],
                sd[b])
        for b in range(2):
            pltpu.make_async_copy(
                zb[b], unf_hbm.at[pl.ds(0, _ZB)], sd[b]).wait()

        @pl.when(sid == NS - 1)
        def _():
            pltpu.sync_copy(acc.at[pl.ds(NS * _ZR, _ZTAIL)],
                            zb0.at[pl.ds(0, _ZTAIL)])
            pltpu.sync_copy(zb0.at[pl.ds(0, _ZTAIL)],
                            unf_hbm.at[pl.ds(plane * N + NS * _ZR, _ZTAIL)])
        pltpu.sync_copy(z128_hbm.at[pl.ds(0, _ZB)], zb0)

    for rep in range(2):
        chunk = cid * 2 + rep
        ebase = chunk * E + sid * _EPT
        zero_acc()
        plsc.subcore_barrier()

        def r_issue(i, b):
            eoff = sid * _EPT + i * _SB
            pltpu.async_copy(dst_hbm.at[pl.ds(eoff, _SB)], ib[b], sr[b])
            pltpu.async_copy(
                msgf_hbm.at[pl.ds(ebase + i * _SB, _SB)], vb[b], sr[b])

        def r_wait(b):
            pltpu.make_async_copy(dst_hbm.at[pl.ds(0, _SB)], ib[b],
                                  sr[b]).wait()
            pltpu.make_async_copy(
                msgf_hbm.at[pl.ds(0, _SB)], vb[b], sr[b]).wait()

        r_issue(0, 0)
        r_issue(1, 1)

        def pair(k, carry):
            for b in range(2):
                i = 2 * k + b
                r_wait(b)
                pltpu.async_copy(vb[b], acc.at[ib[b]], sa[b], add=True)

                @pl.when(i + 2 <= _SNI - 1)
                def _():
                    pltpu.make_async_copy(vb[b], acc.at[ib[b]],
                                          sa[b]).wait()
                    r_issue(i + 2, b)
            return carry

        lax.fori_loop(0, (_SNI - 1) // 2, pair, 0)
        r_wait(0)
        pltpu.async_copy(vb[0], acc.at[ib[0]], sa[0], add=True)
        pltpu.make_async_copy(vb[0], acc.at[ib[0]], sa[0]).wait()
        pltpu.make_async_copy(vb[1], acc.at[ib[1]], sa[1]).wait()
        plsc.subcore_barrier()
        drain(chunk)
        plsc.subcore_barrier()

    zero_acc()
    plsc.subcore_barrier()
    ebase4 = 4 * E + cid * (E // 2) + sid * (_EPT // 2)

    eibase4 = cid * (E // 2) + sid * (_EPT // 2)

    def r4_issue(i, b):
        pltpu.async_copy(dst_hbm.at[pl.ds(eibase4 + i * _SB4, _SB4)],
                         ib4[b], sr[b])
        pltpu.async_copy(
            msgf_hbm.at[pl.ds(ebase4 + i * _SB4, _SB4)],
            vb[b].at[pl.ds(0, _SB4)], sr[b])

    def r4_wait(b):
        pltpu.make_async_copy(dst_hbm.at[pl.ds(0, _SB4)], ib4[b],
                              sr[b]).wait()
        pltpu.make_async_copy(
            msgf_hbm.at[pl.ds(0, _SB4)], vb[b].at[pl.ds(0, _SB4)],
            sr[b]).wait()

    r4_issue(0, 0)
    r4_issue(1, 1)

    def pair4(k, carry):
        for b in range(2):
            i = 2 * k + b
            r4_wait(b)
            pltpu.async_copy(vb[b].at[pl.ds(0, _SB4)], acc.at[ib4[b]],
                             sa[b], add=True)

            @pl.when(i + 2 <= _SNI4 - 1)
            def _():
                pltpu.make_async_copy(vb[b].at[pl.ds(0, _SB4)],
                                      acc.at[ib4[b]], sa[b]).wait()
                r4_issue(i + 2, b)
        return carry

    lax.fori_loop(0, (_SNI4 - 1) // 2, pair4, 0)
    r4_wait(0)
    pltpu.async_copy(vb[0].at[pl.ds(0, _SB4)], acc.at[ib4[0]], sa[0],
                     add=True)
    pltpu.make_async_copy(vb[0].at[pl.ds(0, _SB4)], acc.at[ib4[0]],
                          sa[0]).wait()
    pltpu.make_async_copy(vb[1].at[pl.ds(0, _SB4)], acc.at[ib4[1]],
                          sa[1]).wait()
    plsc.subcore_barrier()
    drain(4 + cid)


@functools.cache
def _scatter():
    return pl.kernel(
        _scatter_body,
        out_type=jax.ShapeDtypeStruct((6 * N, 128), jnp.float32),
        mesh=_sc_mesh(),
        scratch_types=[
            pltpu.VMEM_SHARED((N, 128), jnp.float32),
            pltpu.VMEM((_SB, 128), jnp.float32),
            pltpu.VMEM((_SB, 128), jnp.float32),
            pltpu.VMEM((_SB,), jnp.int32),
            pltpu.VMEM((_SB,), jnp.int32),
            pltpu.VMEM((_SB4,), jnp.int32),
            pltpu.VMEM((_SB4,), jnp.int32),
            pltpu.VMEM((_ZB, 128), jnp.float32),
            pltpu.VMEM((_ZB, 128), jnp.float32),
        ] + [pltpu.SemaphoreType.DMA] * 7,
    )

_NB = 2000


def _norm_body(u_ref, x_ref, b_ref, o_ref):
    d = u_ref[4] + u_ref[5]
    inv = 1.0 / (d + 1e-16)
    scale = jnp.dot(inv, x_ref[...], preferred_element_type=jnp.float32)
    u = jnp.concatenate([u_ref[c] for c in range(4)], axis=1)
    o_ref[...] = u * scale + b_ref[...]


_norm = pl.pallas_call(
    _norm_body,
    grid=(N // _NB,),
    in_specs=[
        pl.BlockSpec((6, _NB, 128), lambda i: (0, i, 0)),
        pl.BlockSpec((128, F), lambda i: (0, 0)),
        pl.BlockSpec((1, F), lambda i: (0, 0)),
    ],
    out_specs=pl.BlockSpec((_NB, F), lambda i: (i, 0)),
    out_shape=jax.ShapeDtypeStruct((N, F), jnp.float32),
)


def kernel(feats, edge_index, W_src, W_dst, attn, bias):
    fs, fd = _mm(feats, W_src, W_dst)
    src = edge_index[0]
    dst = edge_index[1]
    z128 = jnp.zeros((_ZR, 128), jnp.float32)
    msgf = _fused()(fs, fd, src, dst, attn.reshape(F), z128)
    unf = _scatter()(msgf, dst, z128)

    head = jnp.arange(F, dtype=jnp.int32) // OUT_FEATS
    sel = head[:, None] == jnp.arange(16, dtype=jnp.int32)[None, :]
    x16 = sel.T.astype(jnp.float32)
    x128 = jnp.concatenate([x16, jnp.zeros((112, F), jnp.float32)], axis=0)
    return _norm(unf.reshape(6, N, 128), x128, bias.reshape(1, F))
